# baseline (speedup 1.0000x reference)
import functools
import jax
import jax.numpy as jnp
from jax import lax
from jax.experimental import pallas as pl
from jax.experimental.pallas import tpu as pltpu
from jax.experimental.pallas import tpu_sc as plsc

NC = 2
NS = 16
LANES = 16
NW = NC * NS

SUB = 8
LANE = 128
HC = 4
NB = 3
ABLATE = 0


def _make_kernel(BATCH, HIST, D, V):
    b_tiles = BATCH // LANE
    d_tiles = D // SUB
    n_chunks = HIST // HC
    rows_c = HC * LANE
    bg_n = LANE // LANES
    n_main = (n_chunks // NB) * NB - NB
    mesh = plsc.VectorSubcoreMesh(core_axis_name="c", subcore_axis_name="s")

    @functools.partial(
        pl.kernel,
        out_type=jax.ShapeDtypeStruct((HIST, d_tiles, b_tiles, SUB, LANE), jnp.float32),
        mesh=mesh,
        scratch_types=[
            pltpu.VMEM((NB, HC, LANE), jnp.int32),
            pltpu.VMEM((NB, rows_c, D), jnp.float32),
            pltpu.VMEM((NB, HC, D, LANE), jnp.float32),
            pltpu.SemaphoreType.DMA,
            pltpu.SemaphoreType.DMA,
            pltpu.SemaphoreType.DMA,
            pltpu.SemaphoreType.DMA,
            pltpu.SemaphoreType.DMA,
            pltpu.SemaphoreType.DMA,
            pltpu.SemaphoreType.DMA,
            pltpu.SemaphoreType.DMA,
            pltpu.SemaphoreType.DMA,
            pltpu.SemaphoreType.DMA,
            pltpu.SemaphoreType.DMA,
            pltpu.SemaphoreType.DMA,
        ],
        compiler_params=pltpu.CompilerParams(
            use_tc_tiling_on_sc=False, needs_layout_passes=False
        ),
    )
    def k(x_hbm, idx_hbm, table_hbm, out_hbm, idx_v, rows_v, x_v, *sems):
        sI = sems[0:3]
        sG = sems[3:6]
        sX = sems[6:9]
        sO = sems[9:12]
        wid = lax.axis_index("s") * NC + lax.axis_index("c")
        iota = lax.iota(jnp.int32, LANES)
        cols = [jnp.full((LANES,), c, jnp.int32) for c in range(D)]

        def idx_slice(g):
            return idx_hbm.at[g // 2, wid, pl.ds((g % 2) * HC, HC), slice(None)]

        def x_slice(g):
            return x_hbm.at[pl.ds(g * HC, HC), slice(None), wid]

        def out_slice(g):
            return out_hbm.at[pl.ds(g * HC, HC), slice(None), wid]

        def idx_start(g, r):
            pltpu.async_copy(idx_slice(g), idx_v.at[r], sI[r])

        def idx_wait(g, r):
            pltpu.make_async_copy(idx_slice(g), idx_v.at[r], sI[r]).wait()

        def gather_start(r):
            if ABLATE >= 2:
                return
            for hh in range(HC):
                pltpu.async_copy(
                    table_hbm.at[idx_v.at[r, hh]],
                    rows_v.at[r, pl.ds(hh * LANE, LANE)],
                    sG[r],
                )

        def gather_wait(r):
            if ABLATE >= 2:
                return
            pltpu.make_async_copy(
                table_hbm.at[pl.ds(0, rows_c), slice(None)], rows_v.at[r], sG[r]
            ).wait()

        def x_dt_slice(g, dt):
            return x_hbm.at[pl.ds(g * HC, HC), dt, wid]

        def xv_dt_slice(r, dt):
            return x_v.at[r, slice(None), pl.ds(dt * SUB, SUB), slice(None)]

        def x_start(g, r):
            for dt in range(d_tiles):
                pltpu.async_copy(x_dt_slice(g, dt), xv_dt_slice(r, dt), sX[r])

        def x_wait(g, r):
            for dt in range(d_tiles):
                pltpu.make_async_copy(
                    x_dt_slice(g, dt), xv_dt_slice(r, dt), sX[r]
                ).wait()

        def out_start(g, r):
            for dt in range(d_tiles):
                pltpu.async_copy(
                    xv_dt_slice(r, dt),
                    out_hbm.at[pl.ds(g * HC, HC), dt, wid],
                    sO[r],
                )

        def out_wait(g, r):
            for dt in range(d_tiles):
                pltpu.make_async_copy(
                    xv_dt_slice(r, dt),
                    out_hbm.at[pl.ds(g * HC, HC), dt, wid],
                    sO[r],
                ).wait()

        def add(r):
            @plsc.parallel_loop(0, HC * bg_n, 1, unroll=1)
            def _(g):
                rvec = iota + g * LANES
                bvec = iota + (g & 7) * LANES
                hh = g >> 3
                for d0 in range(D):
                    dv = (iota + d0) & (D - 1)
                    v = plsc.load_gather(rows_v.at[r], [rvec, dv])
                    plsc.addupdate_scatter(x_v.at[r, hh], [dv, bvec], v)

        idx_start(0, 0)
        idx_start(1, 1)
        idx_start(2, 2)
        idx_wait(0, 0)
        gather_start(0)
        x_start(0, 0)
        idx_wait(1, 1)
        gather_start(1)
        x_start(1, 1)

        def loop_body(j, carry):
            for c in range(NB):
                g = j * NB + c
                r = c
                x_wait(g, r)
                gather_wait(r)
                if ABLATE < 1:
                    add(r)
                out_start(g, r)

                if c < 2:
                    idx_start(g + NB, r)
                else:
                    @pl.when(j < (n_chunks - 2) // NB - 1)
                    def _():
                        idx_start(g + NB, r)

                r2 = (c + 2) % NB
                idx_wait(g + 2, r2)
                gather_start(r2)

                @pl.when(g >= 1)
                def _():
                    out_wait(g - 1, r2)

                x_start(g + 2, r2)
            return carry

        lax.fori_loop(0, (n_chunks - 2) // NB, loop_body, 0)

        for g in (n_chunks - 2, n_chunks - 1):
            r = g % NB
            x_wait(g, r)
            gather_wait(r)
            if ABLATE < 1:
                add(r)
            out_start(g, r)

        for g in (n_chunks - 3, n_chunks - 2, n_chunks - 1):
            out_wait(g, g % NB)

    return k


def kernel(x, time_index_matrix, embed_weight):
    BATCH, HIST, D = x.shape
    V = embed_weight.shape[0]
    b_tiles = BATCH // LANE
    d_tiles = D // SUB
    h_tiles = HIST // SUB

    x5 = (
        jnp.transpose(x, (1, 2, 0))
        .reshape(HIST, d_tiles, SUB, b_tiles, LANE)
        .transpose(0, 1, 3, 2, 4)
    )
    idx5 = (
        jnp.transpose(time_index_matrix.astype(jnp.int32), (1, 0))
        .reshape(h_tiles, SUB, b_tiles, LANE)
        .transpose(0, 2, 1, 3)
    )

    out5 = _make_kernel(BATCH, HIST, D, V)(x5, idx5, embed_weight)

    out = jnp.transpose(
        out5.transpose(0, 1, 3, 2, 4).reshape(HIST, D, BATCH), (2, 0, 1)
    )
    return out


# speedup vs baseline: 1.0992x; 1.0992x over previous
import functools
import jax
import jax.numpy as jnp
from jax import lax
from jax.experimental import pallas as pl
from jax.experimental.pallas import tpu as pltpu
from jax.experimental.pallas import tpu_sc as plsc

NC = 2
NS = 16
LANES = 16
NW = NC * NS

SUB = 8
LANE = 128
HC = 4
NB = 3
ABLATE = 0


def _make_kernel(BATCH, HIST, D, V):
    b_tiles = BATCH // LANE
    d_tiles = D // SUB
    n_chunks = HIST // HC
    rows_c = HC * LANE
    bg_n = LANE // LANES
    n_main = (n_chunks // NB) * NB - NB
    mesh = plsc.VectorSubcoreMesh(core_axis_name="c", subcore_axis_name="s")

    @functools.partial(
        pl.kernel,
        out_type=jax.ShapeDtypeStruct((HIST, d_tiles, b_tiles, SUB, LANE), jnp.float32),
        mesh=mesh,
        scratch_types=[
            pltpu.VMEM((NB, HC, LANE), jnp.int32),
            pltpu.VMEM((NB, rows_c, D), jnp.float32),
            pltpu.VMEM((NB, HC, D, LANE), jnp.float32),
            pltpu.SemaphoreType.DMA,
            pltpu.SemaphoreType.DMA,
            pltpu.SemaphoreType.DMA,
            pltpu.SemaphoreType.DMA,
            pltpu.SemaphoreType.DMA,
            pltpu.SemaphoreType.DMA,
            pltpu.SemaphoreType.DMA,
            pltpu.SemaphoreType.DMA,
            pltpu.SemaphoreType.DMA,
            pltpu.SemaphoreType.DMA,
            pltpu.SemaphoreType.DMA,
            pltpu.SemaphoreType.DMA,
        ],
        compiler_params=pltpu.CompilerParams(
            use_tc_tiling_on_sc=False, needs_layout_passes=False
        ),
    )
    def k(x_hbm, idx_hbm, table_hbm, out_hbm, idx_v, rows_v, x_v, *sems):
        sI = sems[0:3]
        sG = sems[3:6]
        sX = sems[6:9]
        sO = sems[9:12]
        wid = lax.axis_index("s") * NC + lax.axis_index("c")
        iota = lax.iota(jnp.int32, LANES)
        cols = [jnp.full((LANES,), c, jnp.int32) for c in range(D)]

        def idx_slice(g):
            return idx_hbm.at[g // 2, wid, pl.ds((g % 2) * HC, HC), slice(None)]

        def x_slice(g):
            return x_hbm.at[pl.ds(g * HC, HC), slice(None), wid]

        def out_slice(g):
            return out_hbm.at[pl.ds(g * HC, HC), slice(None), wid]

        def idx_start(g, r):
            pltpu.async_copy(idx_slice(g), idx_v.at[r], sI[r])

        def idx_wait(g, r):
            pltpu.make_async_copy(idx_slice(g), idx_v.at[r], sI[r]).wait()

        def gather_start(r):
            if ABLATE >= 2:
                return
            for hh in range(HC):
                for v in range(LANE // LANES):
                    sl = pl.ds(v * LANES, LANES)
                    idx_v[r, hh, sl] = idx_v[r, hh, sl] * 4
            for hh in range(HC):
                pltpu.async_copy(
                    table_hbm.at[idx_v.at[r, hh]],
                    rows_v.at[r, pl.ds(hh * LANE, LANE)],
                    sG[r],
                )

        def gather_wait(r):
            if ABLATE >= 2:
                return
            pltpu.make_async_copy(
                table_hbm.at[pl.ds(0, rows_c), slice(None)], rows_v.at[r], sG[r]
            ).wait()

        def x_dt_slice(g, dt):
            return x_hbm.at[pl.ds(g * HC, HC), dt, wid]

        def xv_dt_slice(r, dt):
            return x_v.at[r, slice(None), pl.ds(dt * SUB, SUB), slice(None)]

        def x_start(g, r):
            for dt in range(d_tiles):
                pltpu.async_copy(x_dt_slice(g, dt), xv_dt_slice(r, dt), sX[r])

        def x_wait(g, r):
            for dt in range(d_tiles):
                pltpu.make_async_copy(
                    x_dt_slice(g, dt), xv_dt_slice(r, dt), sX[r]
                ).wait()

        def out_start(g, r):
            for dt in range(d_tiles):
                pltpu.async_copy(
                    xv_dt_slice(r, dt),
                    out_hbm.at[pl.ds(g * HC, HC), dt, wid],
                    sO[r],
                )

        def out_wait(g, r):
            for dt in range(d_tiles):
                pltpu.make_async_copy(
                    xv_dt_slice(r, dt),
                    out_hbm.at[pl.ds(g * HC, HC), dt, wid],
                    sO[r],
                ).wait()

        def add(r):
            @plsc.parallel_loop(0, HC * bg_n, 1, unroll=1)
            def _(g):
                rvec = iota + g * LANES
                bvec = iota + (g & 7) * LANES
                hh = g >> 3
                for d0 in range(D):
                    dv = (iota + d0) & (D - 1)
                    v = plsc.load_gather(rows_v.at[r], [rvec, dv])
                    plsc.addupdate_scatter(x_v.at[r, hh], [dv, bvec], v)

        idx_start(0, 0)
        idx_start(1, 1)
        idx_start(2, 2)
        idx_wait(0, 0)
        gather_start(0)
        x_start(0, 0)
        idx_wait(1, 1)
        gather_start(1)
        x_start(1, 1)

        def loop_body(j, carry):
            for c in range(NB):
                g = j * NB + c
                r = c
                x_wait(g, r)
                gather_wait(r)
                if ABLATE < 1:
                    add(r)
                out_start(g, r)

                if c < 2:
                    idx_start(g + NB, r)
                else:
                    @pl.when(j < (n_chunks - 2) // NB - 1)
                    def _():
                        idx_start(g + NB, r)

                r2 = (c + 2) % NB
                idx_wait(g + 2, r2)
                gather_start(r2)

                @pl.when(g >= 1)
                def _():
                    out_wait(g - 1, r2)

                x_start(g + 2, r2)
            return carry

        lax.fori_loop(0, (n_chunks - 2) // NB, loop_body, 0)

        for g in (n_chunks - 2, n_chunks - 1):
            r = g % NB
            x_wait(g, r)
            gather_wait(r)
            if ABLATE < 1:
                add(r)
            out_start(g, r)

        for g in (n_chunks - 3, n_chunks - 2, n_chunks - 1):
            out_wait(g, g % NB)

    return k


def _make_table_transpose(V, D):
    BL = 2048
    grid = (V + BL - 1) // BL

    def body(t_ref, o_ref):
        o_ref[:, 0:D] = t_ref[...].T

    return pl.pallas_call(
        body,
        grid=(grid,),
        in_specs=[pl.BlockSpec((D, BL), lambda i: (0, i))],
        out_specs=pl.BlockSpec((BL, 128), lambda i: (i, 0)),
        out_shape=jax.ShapeDtypeStruct((V, 128), jnp.float32),
    )


def kernel(x, time_index_matrix, embed_weight):
    BATCH, HIST, D = x.shape
    V = embed_weight.shape[0]
    b_tiles = BATCH // LANE
    d_tiles = D // SUB
    h_tiles = HIST // SUB

    x5 = (
        jnp.transpose(x, (1, 2, 0))
        .reshape(HIST, d_tiles, SUB, b_tiles, LANE)
        .transpose(0, 1, 3, 2, 4)
    )
    idx5 = (
        jnp.transpose(time_index_matrix.astype(jnp.int32), (1, 0))
        .reshape(h_tiles, SUB, b_tiles, LANE)
        .transpose(0, 2, 1, 3)
    )

    tt = jnp.transpose(embed_weight, (1, 0))
    table_rm = _make_table_transpose(V, D)(tt).reshape(V * 128 // D, D)

    out5 = _make_kernel(BATCH, HIST, D, V)(x5, idx5, table_rm)

    out = jnp.transpose(
        out5.transpose(0, 1, 3, 2, 4).reshape(HIST, D, BATCH), (2, 0, 1)
    )
    return out


# speedup vs baseline: 1.1339x; 1.0316x over previous
import functools
import jax
import jax.numpy as jnp
from jax import lax
from jax.experimental import pallas as pl
from jax.experimental.pallas import tpu as pltpu
from jax.experimental.pallas import tpu_sc as plsc

NC = 2
NS = 16
LANES = 16
NW = NC * NS

SUB = 8
LANE = 128
HC = 4
NB = 3
ABLATE = 0


def _make_kernel(BATCH, HIST, D, V):
    b_tiles = BATCH // LANE
    d_tiles = D // SUB
    n_chunks = HIST // HC
    rows_c = HC * LANE
    bg_n = LANE // LANES
    n_main = (n_chunks // NB) * NB - NB
    mesh = plsc.VectorSubcoreMesh(core_axis_name="c", subcore_axis_name="s")

    @functools.partial(
        pl.kernel,
        out_type=jax.ShapeDtypeStruct((HIST, d_tiles, b_tiles, SUB, LANE), jnp.float32),
        mesh=mesh,
        scratch_types=[
            pltpu.VMEM((NB, HC, LANE), jnp.int32),
            pltpu.VMEM((NB, rows_c, D), jnp.float32),
            pltpu.VMEM((NB, HC, D, LANE), jnp.float32),
            pltpu.SemaphoreType.DMA,
            pltpu.SemaphoreType.DMA,
            pltpu.SemaphoreType.DMA,
            pltpu.SemaphoreType.DMA,
            pltpu.SemaphoreType.DMA,
            pltpu.SemaphoreType.DMA,
            pltpu.SemaphoreType.DMA,
            pltpu.SemaphoreType.DMA,
            pltpu.SemaphoreType.DMA,
            pltpu.SemaphoreType.DMA,
            pltpu.SemaphoreType.DMA,
            pltpu.SemaphoreType.DMA,
        ],
        compiler_params=pltpu.CompilerParams(
            use_tc_tiling_on_sc=False, needs_layout_passes=False
        ),
    )
    def k(x_hbm, idx_hbm, table_hbm, out_hbm, idx_v, rows_v, x_v, *sems):
        sI = sems[0:3]
        sG = sems[3:6]
        sX = sems[6:9]
        sO = sems[9:12]
        wid = lax.axis_index("s") * NC + lax.axis_index("c")
        iota = lax.iota(jnp.int32, LANES)
        cols = [jnp.full((LANES,), c, jnp.int32) for c in range(D)]

        def idx_slice(g):
            return idx_hbm.at[g // 2, wid, pl.ds((g % 2) * HC, HC), slice(None)]

        def x_slice(g):
            return x_hbm.at[pl.ds(g * HC, HC), slice(None), wid]

        def out_slice(g):
            return out_hbm.at[pl.ds(g * HC, HC), slice(None), wid]

        def idx_start(g, r):
            pltpu.async_copy(idx_slice(g), idx_v.at[r], sI[r])

        def idx_wait(g, r):
            pltpu.make_async_copy(idx_slice(g), idx_v.at[r], sI[r]).wait()

        def gather_start(r):
            if ABLATE >= 2:
                return
            for hh in range(HC):
                for v in range(LANE // LANES):
                    sl = pl.ds(v * LANES, LANES)
                    iv = idx_v[r, hh, sl]
                    t = iv & 2047
                    idx_v[r, hh, sl] = (iv - t) + ((t & 511) << 2) + (t >> 9)
            for hh in range(HC):
                pltpu.async_copy(
                    table_hbm.at[idx_v.at[r, hh]],
                    rows_v.at[r, pl.ds(hh * LANE, LANE)],
                    sG[r],
                )

        def gather_wait(r):
            if ABLATE >= 2:
                return
            pltpu.make_async_copy(
                table_hbm.at[pl.ds(0, rows_c), slice(None)], rows_v.at[r], sG[r]
            ).wait()

        def x_dt_slice(g, dt):
            return x_hbm.at[pl.ds(g * HC, HC), dt, wid]

        def xv_dt_slice(r, dt):
            return x_v.at[r, slice(None), pl.ds(dt * SUB, SUB), slice(None)]

        def x_start(g, r):
            for dt in range(d_tiles):
                pltpu.async_copy(x_dt_slice(g, dt), xv_dt_slice(r, dt), sX[r])

        def x_wait(g, r):
            for dt in range(d_tiles):
                pltpu.make_async_copy(
                    x_dt_slice(g, dt), xv_dt_slice(r, dt), sX[r]
                ).wait()

        def out_start(g, r):
            for dt in range(d_tiles):
                pltpu.async_copy(
                    xv_dt_slice(r, dt),
                    out_hbm.at[pl.ds(g * HC, HC), dt, wid],
                    sO[r],
                )

        def out_wait(g, r):
            for dt in range(d_tiles):
                pltpu.make_async_copy(
                    xv_dt_slice(r, dt),
                    out_hbm.at[pl.ds(g * HC, HC), dt, wid],
                    sO[r],
                ).wait()

        def add(r):
            @plsc.parallel_loop(0, HC * bg_n, 1, unroll=1)
            def _(g):
                rvec = iota + g * LANES
                bvec = iota + (g & 7) * LANES
                hh = g >> 3
                for d0 in range(D):
                    dv = (iota + d0) & (D - 1)
                    v = plsc.load_gather(rows_v.at[r], [rvec, dv])
                    plsc.addupdate_scatter(x_v.at[r, hh], [dv, bvec], v)

        idx_start(0, 0)
        idx_start(1, 1)
        idx_start(2, 2)
        idx_wait(0, 0)
        gather_start(0)
        x_start(0, 0)
        idx_wait(1, 1)
        gather_start(1)
        x_start(1, 1)

        def loop_body(j, carry):
            for c in range(NB):
                g = j * NB + c
                r = c
                x_wait(g, r)
                gather_wait(r)
                if ABLATE < 1:
                    add(r)
                out_start(g, r)

                if c < 2:
                    idx_start(g + NB, r)
                else:
                    @pl.when(j < (n_chunks - 2) // NB - 1)
                    def _():
                        idx_start(g + NB, r)

                r2 = (c + 2) % NB
                idx_wait(g + 2, r2)
                gather_start(r2)

                @pl.when(g >= 1)
                def _():
                    out_wait(g - 1, r2)

                x_start(g + 2, r2)
            return carry

        lax.fori_loop(0, (n_chunks - 2) // NB, loop_body, 0)

        for g in (n_chunks - 2, n_chunks - 1):
            r = g % NB
            x_wait(g, r)
            gather_wait(r)
            if ABLATE < 1:
                add(r)
            out_start(g, r)

        for g in (n_chunks - 3, n_chunks - 2, n_chunks - 1):
            out_wait(g, g % NB)

    return k


def _make_table_transpose(V, D):
    BL = 2048
    grid = (V + BL - 1) // BL

    per = 128 // D
    S = BL // per

    def body(t_ref, o_ref):
        t = t_ref[...].T
        o_ref[...] = jnp.concatenate(
            [t[q * S:(q + 1) * S, :] for q in range(per)], axis=1
        )

    return pl.pallas_call(
        body,
        grid=(grid,),
        in_specs=[pl.BlockSpec((D, BL), lambda i: (0, i))],
        out_specs=pl.BlockSpec((S, 128), lambda i: (i, 0)),
        out_shape=jax.ShapeDtypeStruct((grid * S, 128), jnp.float32),
    )


def kernel(x, time_index_matrix, embed_weight):
    BATCH, HIST, D = x.shape
    V = embed_weight.shape[0]
    b_tiles = BATCH // LANE
    d_tiles = D // SUB
    h_tiles = HIST // SUB

    x5 = (
        jnp.transpose(x, (1, 2, 0))
        .reshape(HIST, d_tiles, SUB, b_tiles, LANE)
        .transpose(0, 1, 3, 2, 4)
    )
    idx5 = (
        jnp.transpose(time_index_matrix.astype(jnp.int32), (1, 0))
        .reshape(h_tiles, SUB, b_tiles, LANE)
        .transpose(0, 2, 1, 3)
    )

    tt = jnp.transpose(embed_weight, (1, 0))
    t128 = _make_table_transpose(V, D)(tt)
    table_rm = t128.reshape(t128.shape[0] * 128 // D, D)

    out5 = _make_kernel(BATCH, HIST, D, V)(x5, idx5, table_rm)

    out = jnp.transpose(
        out5.transpose(0, 1, 3, 2, 4).reshape(HIST, D, BATCH), (2, 0, 1)
    )
    return out


# speedup vs baseline: 1.5981x; 1.4094x over previous
import functools
import jax
import jax.numpy as jnp
from jax import lax
from jax.experimental import pallas as pl
from jax.experimental.pallas import tpu as pltpu
from jax.experimental.pallas import tpu_sc as plsc

NC = 2
NS = 16
LANES = 16
NW = NC * NS

SUB = 8
LANE = 128
HC = 4
NB = 3
ABLATE = 0

TBL = 8192
TS = TBL // 4
TSH = TS.bit_length() - 1


def _make_kernel(BATCH, HIST, D, V):
    b_tiles = BATCH // LANE
    d_tiles = D // SUB
    n_chunks = HIST // HC
    rows_c = HC * LANE
    bg_n = LANE // LANES
    n_main = (n_chunks // NB) * NB - NB
    mesh = plsc.VectorSubcoreMesh(core_axis_name="c", subcore_axis_name="s")

    @functools.partial(
        pl.kernel,
        out_type=jax.ShapeDtypeStruct((HIST, d_tiles, b_tiles, SUB, LANE), jnp.float32),
        mesh=mesh,
        scratch_types=[
            pltpu.VMEM((NB, HC, LANE), jnp.int32),
            pltpu.VMEM((NB, rows_c, D), jnp.float32),
            pltpu.VMEM((NB, HC, D, LANE), jnp.float32),
            pltpu.SemaphoreType.DMA,
            pltpu.SemaphoreType.DMA,
            pltpu.SemaphoreType.DMA,
            pltpu.SemaphoreType.DMA,
            pltpu.SemaphoreType.DMA,
            pltpu.SemaphoreType.DMA,
            pltpu.SemaphoreType.DMA,
            pltpu.SemaphoreType.DMA,
            pltpu.SemaphoreType.DMA,
            pltpu.SemaphoreType.DMA,
            pltpu.SemaphoreType.DMA,
            pltpu.SemaphoreType.DMA,
        ],
        compiler_params=pltpu.CompilerParams(
            use_tc_tiling_on_sc=False, needs_layout_passes=False
        ),
    )
    def k(x_hbm, idx_hbm, table_hbm, out_hbm, idx_v, rows_v, x_v, *sems):
        sI = sems[0:3]
        sG = sems[3:6]
        sX = sems[6:9]
        sO = sems[9:12]
        wid = lax.axis_index("s") * NC + lax.axis_index("c")
        iota = lax.iota(jnp.int32, LANES)
        cols = [jnp.full((LANES,), c, jnp.int32) for c in range(D)]

        def idx_slice(g):
            return idx_hbm.at[g // 2, wid, pl.ds((g % 2) * HC, HC), slice(None)]

        def x_slice(g):
            return x_hbm.at[pl.ds(g * HC, HC), slice(None), wid]

        def out_slice(g):
            return out_hbm.at[pl.ds(g * HC, HC), slice(None), wid]

        def idx_start(g, r):
            pltpu.async_copy(idx_slice(g), idx_v.at[r], sI[r])

        def idx_wait(g, r):
            pltpu.make_async_copy(idx_slice(g), idx_v.at[r], sI[r]).wait()

        def gather_start(r):
            if ABLATE >= 2:
                return
            for hh in range(HC):
                for v in range(LANE // LANES):
                    sl = pl.ds(v * LANES, LANES)
                    iv = idx_v[r, hh, sl]
                    t = iv & (TBL - 1)
                    idx_v[r, hh, sl] = (iv - t) + ((t & (TS - 1)) << 2) + (t >> TSH)
            for hh in range(HC):
                pltpu.async_copy(
                    table_hbm.at[idx_v.at[r, hh]],
                    rows_v.at[r, pl.ds(hh * LANE, LANE)],
                    sG[r],
                )

        def gather_wait(r):
            if ABLATE >= 2:
                return
            pltpu.make_async_copy(
                table_hbm.at[pl.ds(0, rows_c), slice(None)], rows_v.at[r], sG[r]
            ).wait()

        def x_dt_slice(g, dt):
            return x_hbm.at[pl.ds(g * HC, HC), dt, wid]

        def xv_dt_slice(r, dt):
            return x_v.at[r, slice(None), pl.ds(dt * SUB, SUB), slice(None)]

        def x_start(g, r):
            for dt in range(d_tiles):
                pltpu.async_copy(x_dt_slice(g, dt), xv_dt_slice(r, dt), sX[r])

        def x_wait(g, r):
            for dt in range(d_tiles):
                pltpu.make_async_copy(
                    x_dt_slice(g, dt), xv_dt_slice(r, dt), sX[r]
                ).wait()

        def out_start(g, r):
            for dt in range(d_tiles):
                pltpu.async_copy(
                    xv_dt_slice(r, dt),
                    out_hbm.at[pl.ds(g * HC, HC), dt, wid],
                    sO[r],
                )

        def out_wait(g, r):
            for dt in range(d_tiles):
                pltpu.make_async_copy(
                    xv_dt_slice(r, dt),
                    out_hbm.at[pl.ds(g * HC, HC), dt, wid],
                    sO[r],
                ).wait()

        def add(r):
            @plsc.parallel_loop(0, HC * bg_n, 1, unroll=1)
            def _(g):
                rvec = iota + g * LANES
                bvec = iota + (g & 7) * LANES
                hh = g >> 3
                for d0 in range(D):
                    dv = (iota + d0) & (D - 1)
                    v = plsc.load_gather(rows_v.at[r], [rvec, dv])
                    plsc.addupdate_scatter(x_v.at[r, hh], [dv, bvec], v)

        idx_start(0, 0)
        idx_start(1, 1)
        idx_start(2, 2)
        idx_wait(0, 0)
        gather_start(0)
        x_start(0, 0)
        idx_wait(1, 1)
        gather_start(1)
        x_start(1, 1)

        def loop_body(j, carry):
            for c in range(NB):
                g = j * NB + c
                r = c
                x_wait(g, r)
                gather_wait(r)
                if ABLATE < 1:
                    add(r)
                out_start(g, r)

                if c < 2:
                    idx_start(g + NB, r)
                else:
                    @pl.when(j < (n_chunks - 2) // NB - 1)
                    def _():
                        idx_start(g + NB, r)

                r2 = (c + 2) % NB
                idx_wait(g + 2, r2)
                gather_start(r2)

                @pl.when(g >= 1)
                def _():
                    out_wait(g - 1, r2)

                x_start(g + 2, r2)
            return carry

        lax.fori_loop(0, (n_chunks - 2) // NB, loop_body, 0)

        for g in (n_chunks - 2, n_chunks - 1):
            r = g % NB
            x_wait(g, r)
            gather_wait(r)
            if ABLATE < 1:
                add(r)
            out_start(g, r)

        for g in (n_chunks - 3, n_chunks - 2, n_chunks - 1):
            out_wait(g, g % NB)

    return k


def _make_table_transpose(V, D):
    BL = TBL
    grid = (V + BL - 1) // BL

    per = 128 // D
    S = BL // per

    def body(t_ref, o_ref):
        t = jax.lax.dot_general(
            t_ref[...], jnp.eye(D, dtype=jnp.float32),
            (((0,), (0,)), ((), ())),
        )
        o_ref[...] = jnp.concatenate(
            [t[q * S:(q + 1) * S, :] for q in range(per)], axis=1
        )

    return pl.pallas_call(
        body,
        grid=(grid,),
        in_specs=[pl.BlockSpec((D, BL), lambda i: (0, i))],
        out_specs=pl.BlockSpec((S, 128), lambda i: (i, 0)),
        out_shape=jax.ShapeDtypeStruct((grid * S, 128), jnp.float32),
    )


def kernel(x, time_index_matrix, embed_weight):
    BATCH, HIST, D = x.shape
    V = embed_weight.shape[0]
    b_tiles = BATCH // LANE
    d_tiles = D // SUB
    h_tiles = HIST // SUB

    x5 = (
        jnp.transpose(x, (1, 2, 0))
        .reshape(HIST, d_tiles, SUB, b_tiles, LANE)
        .transpose(0, 1, 3, 2, 4)
    )
    idx5 = (
        jnp.transpose(time_index_matrix.astype(jnp.int32), (1, 0))
        .reshape(h_tiles, SUB, b_tiles, LANE)
        .transpose(0, 2, 1, 3)
    )

    tt = jnp.transpose(embed_weight, (1, 0))
    t128 = _make_table_transpose(V, D)(tt)
    table_rm = t128.reshape(t128.shape[0] * 128 // D, D)

    out5 = _make_kernel(BATCH, HIST, D, V)(x5, idx5, table_rm)

    out = jnp.transpose(
        out5.transpose(0, 1, 3, 2, 4).reshape(HIST, D, BATCH), (2, 0, 1)
    )
    return out


# speedup vs baseline: 1.6016x; 1.0022x over previous
import functools
import jax
import jax.numpy as jnp
from jax import lax
from jax.experimental import pallas as pl
from jax.experimental.pallas import tpu as pltpu
from jax.experimental.pallas import tpu_sc as plsc

NC = 2
NS = 16
LANES = 16
NW = NC * NS

SUB = 8
LANE = 128
HC = 4
NB = 3
ABLATE = 0

TBL = 8192
TS = TBL // 4
TSH = TS.bit_length() - 1


def _make_kernel(BATCH, HIST, D, V):
    b_tiles = BATCH // LANE
    d_tiles = D // SUB
    n_chunks = HIST // HC
    rows_c = HC * LANE
    bg_n = LANE // LANES
    n_main = (n_chunks // NB) * NB - NB
    mesh = plsc.VectorSubcoreMesh(core_axis_name="c", subcore_axis_name="s")

    @functools.partial(
        pl.kernel,
        out_type=jax.ShapeDtypeStruct((HIST, d_tiles, b_tiles, SUB, LANE), jnp.float32),
        mesh=mesh,
        scratch_types=[
            pltpu.VMEM((NB, HC, LANE), jnp.int32),
            pltpu.VMEM((NB, rows_c, D), jnp.float32),
            pltpu.VMEM((NB, HC, D, LANE), jnp.float32),
            pltpu.SemaphoreType.DMA,
            pltpu.SemaphoreType.DMA,
            pltpu.SemaphoreType.DMA,
            pltpu.SemaphoreType.DMA,
            pltpu.SemaphoreType.DMA,
            pltpu.SemaphoreType.DMA,
            pltpu.SemaphoreType.DMA,
            pltpu.SemaphoreType.DMA,
            pltpu.SemaphoreType.DMA,
            pltpu.SemaphoreType.DMA,
            pltpu.SemaphoreType.DMA,
            pltpu.SemaphoreType.DMA,
        ],
        compiler_params=pltpu.CompilerParams(
            use_tc_tiling_on_sc=False, needs_layout_passes=False
        ),
    )
    def k(x_hbm, idx_hbm, table_hbm, out_hbm, idx_v, rows_v, x_v, *sems):
        sI = sems[0:3]
        sG = sems[3:6]
        sX = sems[6:9]
        sO = sems[9:12]
        wid = lax.axis_index("s") * NC + lax.axis_index("c")
        iota = lax.iota(jnp.int32, LANES)
        cols = [jnp.full((LANES,), c, jnp.int32) for c in range(D)]

        def idx_slice(g):
            return idx_hbm.at[g // 2, wid, pl.ds((g % 2) * HC, HC), slice(None)]

        def x_slice(g):
            return x_hbm.at[pl.ds(g * HC, HC), slice(None), wid]

        def out_slice(g):
            return out_hbm.at[pl.ds(g * HC, HC), slice(None), wid]

        def idx_start(g, r):
            pltpu.async_copy(idx_slice(g), idx_v.at[r], sI[r])

        def idx_wait(g, r):
            pltpu.make_async_copy(idx_slice(g), idx_v.at[r], sI[r]).wait()

        def gather_start(r):
            if ABLATE >= 2:
                return
            for hh in range(HC):
                for v in range(LANE // LANES):
                    sl = pl.ds(v * LANES, LANES)
                    iv = idx_v[r, hh, sl]
                    t = iv & (TBL - 1)
                    idx_v[r, hh, sl] = (iv - t) + ((t & (TS - 1)) << 2) + (t >> TSH)
            for hh in range(HC):
                pltpu.async_copy(
                    table_hbm.at[idx_v.at[r, hh]],
                    rows_v.at[r, pl.ds(hh * LANE, LANE)],
                    sG[r],
                )

        def gather_wait(r):
            if ABLATE >= 2:
                return
            pltpu.make_async_copy(
                table_hbm.at[pl.ds(0, rows_c), slice(None)], rows_v.at[r], sG[r]
            ).wait()

        def x_dt_slice(g, dt):
            return x_hbm.at[pl.ds(g * HC, HC), dt, wid]

        def xv_dt_slice(r, dt):
            return x_v.at[r, slice(None), pl.ds(dt * SUB, SUB), slice(None)]

        def x_start(g, r):
            for dt in range(d_tiles):
                pltpu.async_copy(x_dt_slice(g, dt), xv_dt_slice(r, dt), sX[r])

        def x_wait(g, r):
            for dt in range(d_tiles):
                pltpu.make_async_copy(
                    x_dt_slice(g, dt), xv_dt_slice(r, dt), sX[r]
                ).wait()

        def out_start(g, r):
            for dt in range(d_tiles):
                pltpu.async_copy(
                    xv_dt_slice(r, dt),
                    out_hbm.at[pl.ds(g * HC, HC), dt, wid],
                    sO[r],
                )

        def out_wait(g, r):
            for dt in range(d_tiles):
                pltpu.make_async_copy(
                    xv_dt_slice(r, dt),
                    out_hbm.at[pl.ds(g * HC, HC), dt, wid],
                    sO[r],
                ).wait()

        def add(r):
            @plsc.parallel_loop(0, HC * bg_n, 1, unroll=1)
            def _(g):
                rvec = iota + g * LANES
                bvec = iota + (g & 7) * LANES
                hh = g >> 3
                for d0 in range(D):
                    dv = (iota + d0) & (D - 1)
                    v = plsc.load_gather(rows_v.at[r], [rvec, dv])
                    plsc.addupdate_scatter(x_v.at[r, hh], [dv, bvec], v)

        idx_start(0, 0)
        idx_start(1, 1)
        idx_start(2, 2)
        idx_wait(0, 0)
        gather_start(0)
        x_start(0, 0)
        idx_wait(1, 1)
        gather_start(1)
        x_start(1, 1)

        def loop_body(j, carry):
            for c in range(NB):
                g = j * NB + c
                r = c
                x_wait(g, r)
                gather_wait(r)
                if ABLATE < 1:
                    add(r)
                out_start(g, r)

                if c < 2:
                    idx_start(g + NB, r)
                else:
                    @pl.when(j < (n_chunks - 2) // NB - 1)
                    def _():
                        idx_start(g + NB, r)

                r2 = (c + 2) % NB
                idx_wait(g + 2, r2)
                gather_start(r2)

                @pl.when(g >= 1)
                def _():
                    out_wait(g - 1, r2)

                x_start(g + 2, r2)
            return carry

        lax.fori_loop(0, (n_chunks - 2) // NB, loop_body, 0)

        for g in (n_chunks - 2, n_chunks - 1):
            r = g % NB
            x_wait(g, r)
            gather_wait(r)
            if ABLATE < 1:
                add(r)
            out_start(g, r)

        for g in (n_chunks - 3, n_chunks - 2, n_chunks - 1):
            out_wait(g, g % NB)

    return k


def _make_table_transpose(V, D):
    BL = TBL
    grid = (V + BL - 1) // BL

    per = 128 // D
    S = BL // per

    def body(t_ref, o_ref):
        t = t_ref[...].T
        o_ref[...] = jnp.concatenate(
            [t[q * S:(q + 1) * S, :] for q in range(per)], axis=1
        )

    return pl.pallas_call(
        body,
        grid=(grid,),
        in_specs=[pl.BlockSpec((D, BL), lambda i: (0, i))],
        out_specs=pl.BlockSpec((S, 128), lambda i: (i, 0)),
        out_shape=jax.ShapeDtypeStruct((grid * S, 128), jnp.float32),
    )


def kernel(x, time_index_matrix, embed_weight):
    BATCH, HIST, D = x.shape
    V = embed_weight.shape[0]
    b_tiles = BATCH // LANE
    d_tiles = D // SUB
    h_tiles = HIST // SUB

    x5 = (
        jnp.transpose(x, (1, 2, 0))
        .reshape(HIST, d_tiles, SUB, b_tiles, LANE)
        .transpose(0, 1, 3, 2, 4)
    )
    idx5 = (
        jnp.transpose(time_index_matrix.astype(jnp.int32), (1, 0))
        .reshape(h_tiles, SUB, b_tiles, LANE)
        .transpose(0, 2, 1, 3)
    )

    tt = jnp.transpose(embed_weight, (1, 0))
    t128 = _make_table_transpose(V, D)(tt)
    table_rm = t128.reshape(t128.shape[0] * 128 // D, D)

    out5 = _make_kernel(BATCH, HIST, D, V)(x5, idx5, table_rm)

    out = jnp.transpose(
        out5.transpose(0, 1, 3, 2, 4).reshape(HIST, D, BATCH), (2, 0, 1)
    )
    return out


# speedup vs baseline: 1.6205x; 1.0118x over previous
import functools
import jax
import jax.numpy as jnp
from jax import lax
from jax.experimental import pallas as pl
from jax.experimental.pallas import tpu as pltpu
from jax.experimental.pallas import tpu_sc as plsc

NC = 2
NS = 16
LANES = 16
NW = NC * NS

SUB = 8
LANE = 128
HC = 4
NB = 3
ABLATE = 0

TBL = 16384
TS = TBL // 4
TSH = TS.bit_length() - 1


def _make_kernel(BATCH, HIST, D, V):
    b_tiles = BATCH // LANE
    d_tiles = D // SUB
    n_chunks = HIST // HC
    rows_c = HC * LANE
    bg_n = LANE // LANES
    n_main = (n_chunks // NB) * NB - NB
    mesh = plsc.VectorSubcoreMesh(core_axis_name="c", subcore_axis_name="s")

    @functools.partial(
        pl.kernel,
        out_type=jax.ShapeDtypeStruct((HIST, d_tiles, b_tiles, SUB, LANE), jnp.float32),
        mesh=mesh,
        scratch_types=[
            pltpu.VMEM((NB, HC, LANE), jnp.int32),
            pltpu.VMEM((NB, rows_c, D), jnp.float32),
            pltpu.VMEM((NB, HC, D, LANE), jnp.float32),
            pltpu.SemaphoreType.DMA,
            pltpu.SemaphoreType.DMA,
            pltpu.SemaphoreType.DMA,
            pltpu.SemaphoreType.DMA,
            pltpu.SemaphoreType.DMA,
            pltpu.SemaphoreType.DMA,
            pltpu.SemaphoreType.DMA,
            pltpu.SemaphoreType.DMA,
            pltpu.SemaphoreType.DMA,
            pltpu.SemaphoreType.DMA,
            pltpu.SemaphoreType.DMA,
            pltpu.SemaphoreType.DMA,
        ],
        compiler_params=pltpu.CompilerParams(
            use_tc_tiling_on_sc=False, needs_layout_passes=False
        ),
    )
    def k(x_hbm, idx_hbm, table_hbm, out_hbm, idx_v, rows_v, x_v, *sems):
        sI = sems[0:3]
        sG = sems[3:6]
        sX = sems[6:9]
        sO = sems[9:12]
        wid = lax.axis_index("s") * NC + lax.axis_index("c")
        iota = lax.iota(jnp.int32, LANES)
        cols = [jnp.full((LANES,), c, jnp.int32) for c in range(D)]

        def idx_slice(g):
            return idx_hbm.at[g // 2, wid, pl.ds((g % 2) * HC, HC), slice(None)]

        def x_slice(g):
            return x_hbm.at[pl.ds(g * HC, HC), slice(None), wid]

        def out_slice(g):
            return out_hbm.at[pl.ds(g * HC, HC), slice(None), wid]

        def idx_start(g, r):
            pltpu.async_copy(idx_slice(g), idx_v.at[r], sI[r])

        def idx_wait(g, r):
            pltpu.make_async_copy(idx_slice(g), idx_v.at[r], sI[r]).wait()

        def gather_start(r):
            if ABLATE >= 2:
                return
            for hh in range(HC):
                for v in range(LANE // LANES):
                    sl = pl.ds(v * LANES, LANES)
                    iv = idx_v[r, hh, sl]
                    t = iv & (TBL - 1)
                    idx_v[r, hh, sl] = (iv - t) + ((t & (TS - 1)) << 2) + (t >> TSH)
            for hh in range(HC):
                pltpu.async_copy(
                    table_hbm.at[idx_v.at[r, hh]],
                    rows_v.at[r, pl.ds(hh * LANE, LANE)],
                    sG[r],
                )

        def gather_wait(r):
            if ABLATE >= 2:
                return
            pltpu.make_async_copy(
                table_hbm.at[pl.ds(0, rows_c), slice(None)], rows_v.at[r], sG[r]
            ).wait()

        def x_dt_slice(g, dt):
            return x_hbm.at[pl.ds(g * HC, HC), dt, wid]

        def xv_dt_slice(r, dt):
            return x_v.at[r, slice(None), pl.ds(dt * SUB, SUB), slice(None)]

        def x_start(g, r):
            for dt in range(d_tiles):
                pltpu.async_copy(x_dt_slice(g, dt), xv_dt_slice(r, dt), sX[r])

        def x_wait(g, r):
            for dt in range(d_tiles):
                pltpu.make_async_copy(
                    x_dt_slice(g, dt), xv_dt_slice(r, dt), sX[r]
                ).wait()

        def out_start(g, r):
            for dt in range(d_tiles):
                pltpu.async_copy(
                    xv_dt_slice(r, dt),
                    out_hbm.at[pl.ds(g * HC, HC), dt, wid],
                    sO[r],
                )

        def out_wait(g, r):
            for dt in range(d_tiles):
                pltpu.make_async_copy(
                    xv_dt_slice(r, dt),
                    out_hbm.at[pl.ds(g * HC, HC), dt, wid],
                    sO[r],
                ).wait()

        def add(r):
            @plsc.parallel_loop(0, HC * bg_n, 1, unroll=1)
            def _(g):
                rvec = iota + g * LANES
                bvec = iota + (g & 7) * LANES
                hh = g >> 3
                for d0 in range(D):
                    dv = (iota + d0) & (D - 1)
                    v = plsc.load_gather(rows_v.at[r], [rvec, dv])
                    plsc.addupdate_scatter(x_v.at[r, hh], [dv, bvec], v)

        idx_start(0, 0)
        idx_start(1, 1)
        idx_start(2, 2)
        idx_wait(0, 0)
        gather_start(0)
        x_start(0, 0)
        idx_wait(1, 1)
        gather_start(1)
        x_start(1, 1)

        def loop_body(j, carry):
            for c in range(NB):
                g = j * NB + c
                r = c
                x_wait(g, r)
                gather_wait(r)
                if ABLATE < 1:
                    add(r)
                out_start(g, r)

                if c < 2:
                    idx_start(g + NB, r)
                else:
                    @pl.when(j < (n_chunks - 2) // NB - 1)
                    def _():
                        idx_start(g + NB, r)

                r2 = (c + 2) % NB
                idx_wait(g + 2, r2)
                gather_start(r2)

                @pl.when(g >= 1)
                def _():
                    out_wait(g - 1, r2)

                x_start(g + 2, r2)
            return carry

        lax.fori_loop(0, (n_chunks - 2) // NB, loop_body, 0)

        for g in (n_chunks - 2, n_chunks - 1):
            r = g % NB
            x_wait(g, r)
            gather_wait(r)
            if ABLATE < 1:
                add(r)
            out_start(g, r)

        for g in (n_chunks - 3, n_chunks - 2, n_chunks - 1):
            out_wait(g, g % NB)

    return k


def _make_table_transpose(V, D):
    BL = TBL
    grid = (V + BL - 1) // BL

    per = 128 // D
    S = BL // per

    def body(t_ref, o_ref):
        t = t_ref[...].T
        o_ref[...] = jnp.concatenate(
            [t[q * S:(q + 1) * S, :] for q in range(per)], axis=1
        )

    return pl.pallas_call(
        body,
        grid=(grid,),
        in_specs=[pl.BlockSpec((D, BL), lambda i: (0, i))],
        out_specs=pl.BlockSpec((S, 128), lambda i: (i, 0)),
        out_shape=jax.ShapeDtypeStruct((grid * S, 128), jnp.float32),
    )


def kernel(x, time_index_matrix, embed_weight):
    BATCH, HIST, D = x.shape
    V = embed_weight.shape[0]
    b_tiles = BATCH // LANE
    d_tiles = D // SUB
    h_tiles = HIST // SUB

    x5 = (
        jnp.transpose(x, (1, 2, 0))
        .reshape(HIST, d_tiles, SUB, b_tiles, LANE)
        .transpose(0, 1, 3, 2, 4)
    )
    idx5 = (
        jnp.transpose(time_index_matrix.astype(jnp.int32), (1, 0))
        .reshape(h_tiles, SUB, b_tiles, LANE)
        .transpose(0, 2, 1, 3)
    )

    tt = jnp.transpose(embed_weight, (1, 0))
    t128 = _make_table_transpose(V, D)(tt)
    table_rm = t128.reshape(t128.shape[0] * 128 // D, D)

    out5 = _make_kernel(BATCH, HIST, D, V)(x5, idx5, table_rm)

    out = jnp.transpose(
        out5.transpose(0, 1, 3, 2, 4).reshape(HIST, D, BATCH), (2, 0, 1)
    )
    return out
